# baseline (speedup 1.0000x reference)
import functools

import jax
import jax.numpy as jnp
from jax.experimental import pallas as pl
from jax.experimental.pallas import tpu as pltpu

N = 10000
E = 40000
E2 = 120000
NUM_REL = 7
NUM_ANGLE = 8
NUM_GRAPHS = 32
EPS = 1e-5
BLK = 512


def _round_up(x, m):
    return ((x + m - 1) // m) * m


def _mm_kernel(a_ref, b_ref, bias_ref, o_ref, *, relu):
    acc = jnp.dot(a_ref[...], b_ref[...], preferred_element_type=jnp.float32)
    acc = acc + bias_ref[...]
    if relu:
        acc = jnp.maximum(acc, 0.0)
    o_ref[...] = acc


def _matmul(a, b, bias, relu=False, bm=1024):
    m, k = a.shape
    k2, n = b.shape
    assert k == k2
    mp = _round_up(m, bm)
    kp = _round_up(k, 128)
    np_ = _round_up(n, 128)
    a = jnp.pad(a, ((0, mp - m), (0, kp - k)))
    b = jnp.pad(b, ((0, kp - k), (0, np_ - n)))
    bias = jnp.pad(bias, ((0, np_ - n),)).reshape(1, np_)
    out = pl.pallas_call(
        functools.partial(_mm_kernel, relu=relu),
        grid=(mp // bm,),
        in_specs=[
            pl.BlockSpec((bm, kp), lambda i: (i, 0)),
            pl.BlockSpec((kp, np_), lambda i: (0, 0)),
            pl.BlockSpec((1, np_), lambda i: (0, 0)),
        ],
        out_specs=pl.BlockSpec((bm, np_), lambda i: (i, 0)),
        out_shape=jax.ShapeDtypeStruct((mp, np_), jnp.float32),
    )(a, b, bias)
    return out[:m, :n]


def _relmm_kernel(blk_rel_ref, g_ref, w_ref, rw_ref, o_ref):
    del blk_rel_ref
    acc = jnp.dot(g_ref[...], w_ref[0], preferred_element_type=jnp.float32)
    o_ref[...] = acc * rw_ref[...]


def _rel_matmul(g, wstack, row_w, blk_rel):
    p, dk = g.shape
    r, dk2, dn = wstack.shape
    assert dk == dk2 and p % BLK == 0
    grid = (p // BLK,)
    return pl.pallas_call(
        _relmm_kernel,
        grid_spec=pltpu.PrefetchScalarGridSpec(
            num_scalar_prefetch=1,
            grid=grid,
            in_specs=[
                pl.BlockSpec((BLK, dk), lambda i, br: (i, 0)),
                pl.BlockSpec((1, dk, dn), lambda i, br: (br[i], 0, 0)),
                pl.BlockSpec((BLK, 1), lambda i, br: (i, 0)),
            ],
            out_specs=pl.BlockSpec((BLK, dn), lambda i, br: (i, 0)),
        ),
        out_shape=jax.ShapeDtypeStruct((p, dn), jnp.float32),
    )(blk_rel, g, wstack.reshape(r, dk, dn), row_w)


def _sorted_rel_plan(rel, num_rel, n_edges):
    p = (_round_up(n_edges, BLK) // BLK + num_rel) * BLK
    perm = jnp.argsort(rel)
    counts = jnp.bincount(rel, length=num_rel)
    off = jnp.concatenate([jnp.zeros((1,), jnp.int32),
                           jnp.cumsum(counts).astype(jnp.int32)])
    blocks_r = (counts + BLK - 1) // BLK
    pad_off = BLK * jnp.concatenate([jnp.zeros((1,), jnp.int32),
                                     jnp.cumsum(blocks_r).astype(jnp.int32)])
    j = jnp.arange(p, dtype=jnp.int32)
    r_j = jnp.clip(jnp.searchsorted(pad_off, j, side="right") - 1,
                   0, num_rel - 1).astype(jnp.int32)
    k = j - pad_off[r_j]
    valid = k < counts[r_j]
    e_map = perm[jnp.clip(off[r_j] + k, 0, n_edges - 1)]
    e_map = jnp.where(valid, e_map, 0)
    blk_rel = jnp.clip(
        jnp.searchsorted(pad_off, jnp.arange(p // BLK, dtype=jnp.int32) * BLK,
                         side="right") - 1, 0, num_rel - 1).astype(jnp.int32)
    return e_map, valid, blk_rel


def _bn(x, g, b):
    m = jnp.mean(x, axis=0)
    v = jnp.var(x, axis=0)
    return (x - m) / jnp.sqrt(v + EPS) * g + b


def _split_w(linw, num_rel, d_in, dn_pad):
    d_out = linw.shape[1]
    w = linw.reshape(num_rel, d_in, d_out)
    dk_pad = _round_up(d_in, 128)
    return jnp.pad(w, ((0, 0), (0, dk_pad - d_in), (0, dn_pad - d_out)))


def _conv_sorted(x, src, dst_pad, w_pad, valid, blk_rel, e_map, num_rel, p,
                 n_out):
    d_in = x.shape[1]
    d_out = p["linW"].shape[1]
    dn_pad = _round_up(d_out, 128)
    dk_pad = _round_up(d_in, 128)
    gather_idx = jnp.where(valid, src[e_map], 0)
    g = jnp.pad(x, ((0, 0), (0, dk_pad - d_in)))[gather_idx]
    wstack = _split_w(p["linW"], num_rel, d_in, dn_pad)
    m = _rel_matmul(g, wstack, w_pad, blk_rel)
    return jnp.zeros((n_out, d_out), jnp.float32).at[dst_pad].add(m[:, :d_out])


def kernel(node_feature, edge_index, edge_relation, edge_feature, edge_weight,
           line_edge_index, line_edge_relation, line_edge_weight, node2graph,
           params):
    e_map_n, valid_n, blk_rel_n = _sorted_rel_plan(edge_relation, NUM_REL, E)
    e_map_l, valid_l, blk_rel_l = _sorted_rel_plan(line_edge_relation,
                                                   NUM_ANGLE, E2)
    dst_n = jnp.where(valid_n, edge_index[1][e_map_n], 0)
    dst_l = jnp.where(valid_l, line_edge_index[1][e_map_l], 0)
    w_n = jnp.where(valid_n, edge_weight[e_map_n], 0.0)[:, None]
    w_l = jnp.where(valid_l, line_edge_weight[e_map_l], 0.0)[:, None]

    hiddens = []
    layer_input = node_feature
    edge_input = edge_feature
    for i in range(3):
        pn = params["node"][i]
        pe = params["edge"][i]
        s = _conv_sorted(layer_input, edge_index[0], dst_n, w_n, valid_n,
                         blk_rel_n, e_map_n, NUM_REL, pn, N)
        y = s + pn["linb"] + _matmul(layer_input, pn["slW"], pn["slb"])
        hidden = jax.nn.relu(_bn(y, pn["bng"], pn["bnb"]))
        if hidden.shape == layer_input.shape:
            hidden = hidden + layer_input
        s2 = _conv_sorted(edge_input, line_edge_index[0], dst_l, w_l, valid_l,
                          blk_rel_l, e_map_l, NUM_ANGLE, pe, E)
        y2 = s2 + pe["linb"] + _matmul(edge_input, pe["slW"], pe["slb"])
        edge_hidden = jax.nn.relu(_bn(y2, pe["bng"], pe["bnb"]))
        d_eh = edge_hidden.shape[1]
        dn_pad = 512
        g = edge_hidden[jnp.where(valid_n, e_map_n, 0)]
        g = jnp.pad(g, ((0, 0), (0, _round_up(d_eh, 128) - d_eh)))
        wstack = _split_w(pn["linW"], NUM_REL, d_eh, dn_pad)
        m = _rel_matmul(g, wstack, w_n, blk_rel_n)
        upd = jnp.zeros((N, dn_pad), jnp.float32).at[dst_n].add(m)[:, :512]
        upd = jax.nn.relu(upd + pn["linb"])
        hidden = hidden + upd
        edge_input = edge_hidden
        hidden = _bn(hidden, params["bn"][i]["g"], params["bn"][i]["b"])
        hiddens.append(hidden)
        layer_input = hidden
    node_feat = jnp.concatenate(hiddens, axis=-1)
    graph_feat = jax.ops.segment_sum(node_feat, node2graph,
                                     num_segments=NUM_GRAPHS)
    return graph_feat, node_feat
